# baseline (speedup 1.0000x reference)
import functools

import jax
import jax.numpy as jnp
from jax import lax
from jax.experimental import pallas as pl
from jax.experimental.pallas import tpu as pltpu
from jax.experimental.pallas import tpu_sc as plsc


def _stage1_body(x1_ref, x2t_ref, i0_ref, i1_ref, i2_ref, *, blk, s):
    bi = pl.program_id(0)

    x1 = x1_ref[0]
    x2t = x2t_ref[0]
    cross = jnp.dot(x1, x2t, preferred_element_type=jnp.float32)
    n1 = jnp.sum(x1 * x1, axis=1, keepdims=True)
    n2 = jnp.sum(x2t * x2t, axis=0, keepdims=True)
    d = -2.0 * cross
    d = d + n1
    d = d + n2
    d = jnp.maximum(d, jnp.float32(0.0001))

    ch = 128
    nch = s // ch
    m1 = d[:, 0:ch]
    inf = jnp.full((blk, ch), jnp.inf, jnp.float32)
    m2 = inf
    m3 = inf
    for c in range(1, nch):
        cv = d[:, c * ch:(c + 1) * ch]
        nm1 = jnp.minimum(m1, cv)
        pu = jnp.maximum(m1, cv)
        nm2 = jnp.minimum(m2, pu)
        pu2 = jnp.maximum(m2, pu)
        m3 = jnp.minimum(m3, pu2)
        m1, m2 = nm1, nm2
    off = 1
    while off < ch:
        r1 = pltpu.roll(m1, ch - off, 1)
        r2 = pltpu.roll(m2, ch - off, 1)
        r3 = pltpu.roll(m3, ch - off, 1)
        p = jnp.maximum(m1, r1)
        q = jnp.minimum(m2, r2)
        c1 = jnp.minimum(m1, r1)
        c2 = jnp.minimum(p, q)
        c3 = jnp.minimum(jnp.maximum(p, q), jnp.minimum(m3, r3))
        m1, m2, m3 = c1, c2, c3
        off *= 2
    v3 = m3[:, 0:1]

    iota = jax.lax.broadcasted_iota(jnp.int32, (blk, s), 1)
    mask = d <= v3
    key_lo = jnp.where(mask, iota, jnp.int32(s))
    key_hi = jnp.where(mask, iota, jnp.int32(-1))
    i_min = jnp.min(key_lo, axis=1, keepdims=True)
    i_max = jnp.max(key_hi, axis=1, keepdims=True)
    i_sum = jnp.sum(jnp.maximum(key_hi, 0), axis=1, keepdims=True)
    i_mid = jnp.clip(i_sum - i_min - i_max, 0, jnp.int32(s - 1))

    base = bi * jnp.int32(s)
    i0_ref[0] = i_min + base
    i1_ref[0] = i_mid + base
    i2_ref[0] = i_max + base


def _sc_gather(idx0, idx1, idx2, tabf, tabx, *, bn, c2):
    info = plsc.get_sparse_core_info()
    nw = info.num_cores * info.num_subcores
    per_w = bn // nw
    chunk = 128
    nch = per_w // chunk
    mesh = plsc.VectorSubcoreMesh(core_axis_name="c", subcore_axis_name="s")

    @functools.partial(
        pl.kernel, mesh=mesh,
        out_type=[jax.ShapeDtypeStruct((bn, c2), jnp.float32)
                  for _ in range(3)]
                 + [jax.ShapeDtypeStruct((bn, 128), jnp.float32)
                    for _ in range(3)],
        scratch_types=[
            pltpu.VMEM((per_w,), jnp.int32),
            pltpu.VMEM((per_w,), jnp.int32),
            pltpu.VMEM((per_w,), jnp.int32),
            pltpu.VMEM((chunk, c2), jnp.float32),
            pltpu.VMEM((chunk, c2), jnp.float32),
            pltpu.VMEM((chunk, 128), jnp.float32),
            pltpu.VMEM((chunk, 128), jnp.float32),
            pltpu.SemaphoreType.DMA,
            pltpu.SemaphoreType.DMA,
            pltpu.SemaphoreType.DMA,
            pltpu.SemaphoreType.DMA,
            pltpu.SemaphoreType.DMA,
            pltpu.SemaphoreType.DMA,
        ],
    )
    def gather(i0_hbm, i1_hbm, i2_hbm, tf_hbm, tx_hbm,
               f0_hbm, f1_hbm, f2_hbm, x0_hbm, x1_hbm, x2_hbm,
               iv0, iv1, iv2, bf0, bf1, bx0, bx1,
               fsem0, fsem1, xsem0, xsem1, wsem0, wsem1):
        wid = lax.axis_index("s") * info.num_cores + lax.axis_index("c")
        base = wid * per_w
        pltpu.sync_copy(i0_hbm.at[pl.ds(base, per_w)], iv0)
        pltpu.sync_copy(i1_hbm.at[pl.ds(base, per_w)], iv1)
        pltpu.sync_copy(i2_hbm.at[pl.ds(base, per_w)], iv2)

        ivs = (iv0, iv1, iv2)
        fouts = (f0_hbm, f1_hbm, f2_hbm)
        xouts = (x0_hbm, x1_hbm, x2_hbm)
        fbufs = (bf0, bf1)
        xbufs = (bx0, bx1)
        fsems = (fsem0, fsem1)
        xsems = (xsem0, xsem1)
        pairs = [(k_, c_) for k_ in range(3) for c_ in range(nch)]
        npairs = len(pairs)

        def gstart(j):
            k_, c_ = pairs[j]
            idx = ivs[k_].at[pl.ds(c_ * chunk, chunk)]
            return (pltpu.async_copy(tf_hbm.at[idx], fbufs[j % 2],
                                     fsems[j % 2]),
                    pltpu.async_copy(tx_hbm.at[idx], xbufs[j % 2],
                                     xsems[j % 2]))

        wsems = (wsem0, wsem1)
        gcps = [gstart(0), None]
        wcps = [None, None]
        for j in range(npairs):
            gcps[j % 2][0].wait()
            gcps[j % 2][1].wait()
            k_, c_ = pairs[j]
            rows = pl.ds(base + c_ * chunk, chunk)
            wcps[j % 2] = pltpu.async_copy(fbufs[j % 2], fouts[k_].at[rows],
                                           wsems[j % 2])
            pltpu.sync_copy(xbufs[j % 2], xouts[k_].at[rows])
            if j + 1 < npairs:
                if wcps[(j + 1) % 2] is not None:
                    wcps[(j + 1) % 2].wait()
                gcps[(j + 1) % 2] = gstart(j + 1)
        wcps[(npairs - 1) % 2].wait()
        if wcps[npairs % 2] is not None:
            wcps[npairs % 2].wait()

    return gather(idx0, idx1, idx2, tabf, tabx)


def _stage2_body(x1_ref, p1_ref, g0_ref, g1_ref, g2_ref,
                 x0_ref, xx1_ref, x2_ref,
                 w1a_ref, w1b_ref, b1_ref, h1_ref, stats_ref):
    i = pl.program_id(0)
    x1 = x1_ref[...]
    x1b = x1.astype(jnp.bfloat16).astype(jnp.float32)
    n1 = jnp.sum(x1 * x1, axis=1, keepdims=True)

    def dist(xr):
        x2 = xr[...]
        x2b = x2.astype(jnp.bfloat16).astype(jnp.float32)
        cross = jnp.sum(x1b * x2b, axis=1, keepdims=True)
        n2 = jnp.sum(x2 * x2, axis=1, keepdims=True)
        return (-2.0 * cross + n1) + n2

    d0 = jnp.maximum(dist(x0_ref), jnp.float32(0.0001))
    d1 = jnp.maximum(dist(xx1_ref), jnp.float32(0.0001))
    d2 = jnp.maximum(dist(x2_ref), jnp.float32(0.0001))
    r0 = 1.0 / (d0 + jnp.float32(0.0001))
    r1 = 1.0 / (d1 + jnp.float32(0.0001))
    r2 = 1.0 / (d2 + jnp.float32(0.0001))
    scale = 1.0 / (r0 + r1 + r2 + jnp.float32(0.0001))

    interp = (g0_ref[...] * (r0 * scale) + g1_ref[...] * (r1 * scale)
              + g2_ref[...] * (r2 * scale))
    h1 = (jnp.dot(p1_ref[...], w1a_ref[...], preferred_element_type=jnp.float32)
          + jnp.dot(interp, w1b_ref[...], preferred_element_type=jnp.float32)
          + b1_ref[...])
    h1_ref[...] = h1

    @pl.when(i == 0)
    def _init():
        stats_ref[...] = jnp.zeros_like(stats_ref)

    stats_ref[0:1, :] += jnp.sum(h1, axis=0, keepdims=True)
    stats_ref[1:2, :] += jnp.sum(h1 * h1, axis=0, keepdims=True)


def _stage3_body(h1_ref, sc1_ref, sh1_ref, w2_ref, b2_ref,
                 h2_ref, stats_ref):
    i = pl.program_id(0)
    h = h1_ref[...]
    h = jnp.maximum(h * sc1_ref[...] + sh1_ref[...], jnp.float32(0.0))
    h2 = jnp.dot(h, w2_ref[...], preferred_element_type=jnp.float32) + b2_ref[...]
    h2_ref[...] = h2

    @pl.when(i == 0)
    def _init():
        stats_ref[...] = jnp.zeros_like(stats_ref)

    stats_ref[0:1, :] += jnp.sum(h2, axis=0, keepdims=True)
    stats_ref[1:2, :] += jnp.sum(h2 * h2, axis=0, keepdims=True)


def _stage4_body(h2_ref, sc2_ref, sh2_ref, out_ref):
    out_ref[...] = jnp.maximum(
        h2_ref[...] * sc2_ref[...] + sh2_ref[...], jnp.float32(0.0))


def kernel(xyz1, xyz2, points1, points2, W1, b1, g1, be1, W2, b2, g2, be2):
    B, N, _ = xyz1.shape
    S = xyz2.shape[1]
    C1 = points1.shape[2]
    C2 = points2.shape[2]
    C = W1.shape[1]
    BN = B * N

    blk1 = 256 if N % 256 == 0 else N
    blk2 = 512 if BN % 512 == 0 else BN

    xyz1p = jnp.pad(xyz1, ((0, 0), (0, 0), (0, 5)))
    xyz2p = jnp.pad(xyz2, ((0, 0), (0, 0), (0, 5)))
    xyz2t = jnp.transpose(xyz2p, (0, 2, 1))
    W1a = W1[:C1]
    W1b = W1[C1:]
    b1r = b1.reshape(1, C)
    b2r = b2.reshape(1, C)
    tabf = points2.reshape(B * S, C2)
    tabx = jnp.pad(xyz2p, ((0, 0), (0, 0), (0, 120))).reshape(B * S, 128)

    nblk = N // blk1
    i0, i1, i2 = pl.pallas_call(
        functools.partial(_stage1_body, blk=blk1, s=S),
        grid=(B, nblk),
        in_specs=[
            pl.BlockSpec((1, blk1, 8), lambda b, n: (b, n, 0)),
            pl.BlockSpec((1, 8, S), lambda b, n: (b, 0, 0)),
        ],
        out_specs=[pl.BlockSpec((1, blk1, 1), lambda b, n: (b, n, 0))
                   for _ in range(3)],
        out_shape=[jax.ShapeDtypeStruct((B, N, 1), jnp.int32)
                   for _ in range(3)],
        compiler_params=pltpu.CompilerParams(
            dimension_semantics=("arbitrary", "arbitrary")),
    )(xyz1p, xyz2t)

    ga, gb, gc, xa, xb, xc = _sc_gather(
        i0.reshape(BN), i1.reshape(BN), i2.reshape(BN), tabf, tabx,
        bn=BN, c2=C2)

    x1f = jnp.pad(xyz1p, ((0, 0), (0, 0), (0, 120))).reshape(BN, 128)
    p1f = points1.reshape(BN, C1)
    nblk2 = BN // blk2
    cspec = pl.BlockSpec((blk2, C2), lambda i: (i, 0))
    xspec = pl.BlockSpec((blk2, 128), lambda i: (i, 0))
    h1, stats1 = pl.pallas_call(
        _stage2_body,
        grid=(nblk2,),
        in_specs=[
            pl.BlockSpec((blk2, 128), lambda i: (i, 0)),
            pl.BlockSpec((blk2, C1), lambda i: (i, 0)),
            cspec, cspec, cspec, xspec, xspec, xspec,
            pl.BlockSpec((C1, C), lambda i: (0, 0)),
            pl.BlockSpec((C2, C), lambda i: (0, 0)),
            pl.BlockSpec((1, C), lambda i: (0, 0)),
        ],
        out_specs=[
            pl.BlockSpec((blk2, C), lambda i: (i, 0)),
            pl.BlockSpec((8, C), lambda i: (0, 0)),
        ],
        out_shape=[
            jax.ShapeDtypeStruct((BN, C), jnp.float32),
            jax.ShapeDtypeStruct((8, C), jnp.float32),
        ],
        compiler_params=pltpu.CompilerParams(
            dimension_semantics=("arbitrary",)),
    )(x1f, p1f, ga, gb, gc, xa, xb, xc, W1a, W1b, b1r)

    cnt = jnp.float32(BN)
    mean1 = stats1[0:1] / cnt
    var1 = stats1[1:2] / cnt - mean1 * mean1
    sc1v = g1.reshape(1, C) / jnp.sqrt(var1 + 1e-5)
    sh1v = be1.reshape(1, C) - mean1 * sc1v

    h2, stats2 = pl.pallas_call(
        _stage3_body,
        grid=(nblk2,),
        in_specs=[
            pl.BlockSpec((blk2, C), lambda i: (i, 0)),
            pl.BlockSpec((1, C), lambda i: (0, 0)),
            pl.BlockSpec((1, C), lambda i: (0, 0)),
            pl.BlockSpec((C, C), lambda i: (0, 0)),
            pl.BlockSpec((1, C), lambda i: (0, 0)),
        ],
        out_specs=[
            pl.BlockSpec((blk2, C), lambda i: (i, 0)),
            pl.BlockSpec((8, C), lambda i: (0, 0)),
        ],
        out_shape=[
            jax.ShapeDtypeStruct((BN, C), jnp.float32),
            jax.ShapeDtypeStruct((8, C), jnp.float32),
        ],
        compiler_params=pltpu.CompilerParams(
            dimension_semantics=("arbitrary",)),
    )(h1, sc1v, sh1v, W2, b2r)

    mean2 = stats2[0:1] / cnt
    var2 = stats2[1:2] / cnt - mean2 * mean2
    sc2v = g2.reshape(1, C) / jnp.sqrt(var2 + 1e-5)
    sh2v = be2.reshape(1, C) - mean2 * sc2v

    out = pl.pallas_call(
        _stage4_body,
        grid=(nblk2,),
        in_specs=[
            pl.BlockSpec((blk2, C), lambda i: (i, 0)),
            pl.BlockSpec((1, C), lambda i: (0, 0)),
            pl.BlockSpec((1, C), lambda i: (0, 0)),
        ],
        out_specs=pl.BlockSpec((blk2, C), lambda i: (i, 0)),
        out_shape=jax.ShapeDtypeStruct((BN, C), jnp.float32),
    )(h2, sc2v, sh2v)

    return out.reshape(B, N, C)


# speedup vs baseline: 1.0044x; 1.0044x over previous
import functools

import jax
import jax.numpy as jnp
from jax import lax
from jax.experimental import pallas as pl
from jax.experimental.pallas import tpu as pltpu
from jax.experimental.pallas import tpu_sc as plsc


def _stage1_body(x1_ref, x2t_ref, i0_ref, i1_ref, i2_ref, *, blk, s):
    bi = pl.program_id(0)

    x1 = x1_ref[0]
    x2t = x2t_ref[0]
    cross = jnp.dot(x1, x2t, preferred_element_type=jnp.float32)
    n1 = jnp.sum(x1 * x1, axis=1, keepdims=True)
    n2 = jnp.sum(x2t * x2t, axis=0, keepdims=True)
    d = -2.0 * cross
    d = d + n1
    d = d + n2
    d = jnp.maximum(d, jnp.float32(0.0001))

    ch = 128
    nch = s // ch
    m1 = d[:, 0:ch]
    inf = jnp.full((blk, ch), jnp.inf, jnp.float32)
    m2 = inf
    m3 = inf
    for c in range(1, nch):
        cv = d[:, c * ch:(c + 1) * ch]
        nm1 = jnp.minimum(m1, cv)
        pu = jnp.maximum(m1, cv)
        nm2 = jnp.minimum(m2, pu)
        pu2 = jnp.maximum(m2, pu)
        m3 = jnp.minimum(m3, pu2)
        m1, m2 = nm1, nm2
    off = 1
    while off < ch:
        r1 = pltpu.roll(m1, ch - off, 1)
        r2 = pltpu.roll(m2, ch - off, 1)
        r3 = pltpu.roll(m3, ch - off, 1)
        p = jnp.maximum(m1, r1)
        q = jnp.minimum(m2, r2)
        c1 = jnp.minimum(m1, r1)
        c2 = jnp.minimum(p, q)
        c3 = jnp.minimum(jnp.maximum(p, q), jnp.minimum(m3, r3))
        m1, m2, m3 = c1, c2, c3
        off *= 2
    v3 = m3[:, 0:1]

    iota = jax.lax.broadcasted_iota(jnp.int32, (blk, s), 1)
    mask = d <= v3
    key_lo = jnp.where(mask, iota, jnp.int32(s))
    key_hi = jnp.where(mask, iota, jnp.int32(-1))
    i_min = jnp.min(key_lo, axis=1, keepdims=True)
    i_max = jnp.max(key_hi, axis=1, keepdims=True)
    i_sum = jnp.sum(jnp.maximum(key_hi, 0), axis=1, keepdims=True)
    i_mid = jnp.clip(i_sum - i_min - i_max, 0, jnp.int32(s - 1))

    base = bi * jnp.int32(s)
    i0_ref[0] = i_min + base
    i1_ref[0] = i_mid + base
    i2_ref[0] = i_max + base


def _sc_gather(idx0, idx1, idx2, tabf, tabx, *, bn, c2):
    info = plsc.get_sparse_core_info()
    nw = info.num_cores * info.num_subcores
    per_w = bn // nw
    chunk = 128
    nch = per_w // chunk
    mesh = plsc.VectorSubcoreMesh(core_axis_name="c", subcore_axis_name="s")

    @functools.partial(
        pl.kernel, mesh=mesh,
        out_type=[jax.ShapeDtypeStruct((bn, c2), jnp.float32)
                  for _ in range(3)]
                 + [jax.ShapeDtypeStruct((bn, 128), jnp.float32)
                    for _ in range(3)],
        scratch_types=[
            pltpu.VMEM((per_w,), jnp.int32),
            pltpu.VMEM((per_w,), jnp.int32),
            pltpu.VMEM((per_w,), jnp.int32),
            pltpu.VMEM((chunk, c2), jnp.float32),
            pltpu.VMEM((chunk, c2), jnp.float32),
            pltpu.VMEM((chunk, 128), jnp.float32),
            pltpu.VMEM((chunk, 128), jnp.float32),
            pltpu.SemaphoreType.DMA,
            pltpu.SemaphoreType.DMA,
            pltpu.SemaphoreType.DMA,
            pltpu.SemaphoreType.DMA,
        ],
    )
    def gather(i0_hbm, i1_hbm, i2_hbm, tf_hbm, tx_hbm,
               f0_hbm, f1_hbm, f2_hbm, x0_hbm, x1_hbm, x2_hbm,
               iv0, iv1, iv2, bf0, bf1, bx0, bx1,
               fsem0, fsem1, xsem0, xsem1):
        wid = lax.axis_index("s") * info.num_cores + lax.axis_index("c")
        base = wid * per_w
        pltpu.sync_copy(i0_hbm.at[pl.ds(base, per_w)], iv0)
        pltpu.sync_copy(i1_hbm.at[pl.ds(base, per_w)], iv1)
        pltpu.sync_copy(i2_hbm.at[pl.ds(base, per_w)], iv2)

        ivs = (iv0, iv1, iv2)
        fouts = (f0_hbm, f1_hbm, f2_hbm)
        xouts = (x0_hbm, x1_hbm, x2_hbm)
        fbufs = (bf0, bf1)
        xbufs = (bx0, bx1)
        fsems = (fsem0, fsem1)
        xsems = (xsem0, xsem1)
        pairs = [(k_, c_) for k_ in range(3) for c_ in range(nch)]
        npairs = len(pairs)

        def gstart(j):
            k_, c_ = pairs[j]
            idx = ivs[k_].at[pl.ds(c_ * chunk, chunk)]
            return (pltpu.async_copy(tf_hbm.at[idx], fbufs[j % 2],
                                     fsems[j % 2]),
                    pltpu.async_copy(tx_hbm.at[idx], xbufs[j % 2],
                                     xsems[j % 2]))

        gcps = [gstart(0), None]
        for j in range(npairs):
            if j + 1 < npairs:
                gcps[(j + 1) % 2] = gstart(j + 1)
            gcps[j % 2][0].wait()
            gcps[j % 2][1].wait()
            k_, c_ = pairs[j]
            rows = pl.ds(base + c_ * chunk, chunk)
            pltpu.sync_copy(fbufs[j % 2], fouts[k_].at[rows])
            pltpu.sync_copy(xbufs[j % 2], xouts[k_].at[rows])

    return gather(idx0, idx1, idx2, tabf, tabx)


def _stage2_body(x1_ref, p1_ref, g0_ref, g1_ref, g2_ref,
                 x0_ref, xx1_ref, x2_ref,
                 w1a_ref, w1b_ref, b1_ref, h1_ref, stats_ref):
    i = pl.program_id(0)
    x1 = x1_ref[...]
    x1b = x1.astype(jnp.bfloat16).astype(jnp.float32)
    n1 = jnp.sum(x1 * x1, axis=1, keepdims=True)

    def dist(xr):
        x2 = xr[...]
        x2b = x2.astype(jnp.bfloat16).astype(jnp.float32)
        cross = jnp.sum(x1b * x2b, axis=1, keepdims=True)
        n2 = jnp.sum(x2 * x2, axis=1, keepdims=True)
        return (-2.0 * cross + n1) + n2

    d0 = jnp.maximum(dist(x0_ref), jnp.float32(0.0001))
    d1 = jnp.maximum(dist(xx1_ref), jnp.float32(0.0001))
    d2 = jnp.maximum(dist(x2_ref), jnp.float32(0.0001))
    r0 = 1.0 / (d0 + jnp.float32(0.0001))
    r1 = 1.0 / (d1 + jnp.float32(0.0001))
    r2 = 1.0 / (d2 + jnp.float32(0.0001))
    scale = 1.0 / (r0 + r1 + r2 + jnp.float32(0.0001))

    interp = (g0_ref[...] * (r0 * scale) + g1_ref[...] * (r1 * scale)
              + g2_ref[...] * (r2 * scale))
    h1 = (jnp.dot(p1_ref[...], w1a_ref[...], preferred_element_type=jnp.float32)
          + jnp.dot(interp, w1b_ref[...], preferred_element_type=jnp.float32)
          + b1_ref[...])
    h1_ref[...] = h1

    @pl.when(i == 0)
    def _init():
        stats_ref[...] = jnp.zeros_like(stats_ref)

    stats_ref[0:1, :] += jnp.sum(h1, axis=0, keepdims=True)
    stats_ref[1:2, :] += jnp.sum(h1 * h1, axis=0, keepdims=True)


def _stage3_body(h1_ref, sc1_ref, sh1_ref, w2_ref, b2_ref,
                 h2_ref, stats_ref):
    i = pl.program_id(0)
    h = h1_ref[...]
    h = jnp.maximum(h * sc1_ref[...] + sh1_ref[...], jnp.float32(0.0))
    h2 = jnp.dot(h, w2_ref[...], preferred_element_type=jnp.float32) + b2_ref[...]
    h2_ref[...] = h2

    @pl.when(i == 0)
    def _init():
        stats_ref[...] = jnp.zeros_like(stats_ref)

    stats_ref[0:1, :] += jnp.sum(h2, axis=0, keepdims=True)
    stats_ref[1:2, :] += jnp.sum(h2 * h2, axis=0, keepdims=True)


def _stage4_body(h2_ref, sc2_ref, sh2_ref, out_ref):
    out_ref[...] = jnp.maximum(
        h2_ref[...] * sc2_ref[...] + sh2_ref[...], jnp.float32(0.0))


def kernel(xyz1, xyz2, points1, points2, W1, b1, g1, be1, W2, b2, g2, be2):
    B, N, _ = xyz1.shape
    S = xyz2.shape[1]
    C1 = points1.shape[2]
    C2 = points2.shape[2]
    C = W1.shape[1]
    BN = B * N

    blk1 = 256 if N % 256 == 0 else N
    blk2 = 512 if BN % 512 == 0 else BN

    xyz1p = jnp.pad(xyz1, ((0, 0), (0, 0), (0, 5)))
    xyz2p = jnp.pad(xyz2, ((0, 0), (0, 0), (0, 5)))
    xyz2t = jnp.transpose(xyz2p, (0, 2, 1))
    W1a = W1[:C1]
    W1b = W1[C1:]
    b1r = b1.reshape(1, C)
    b2r = b2.reshape(1, C)
    tabf = points2.reshape(B * S, C2)
    tabx = jnp.pad(xyz2p, ((0, 0), (0, 0), (0, 120))).reshape(B * S, 128)

    nblk = N // blk1
    i0, i1, i2 = pl.pallas_call(
        functools.partial(_stage1_body, blk=blk1, s=S),
        grid=(B, nblk),
        in_specs=[
            pl.BlockSpec((1, blk1, 8), lambda b, n: (b, n, 0)),
            pl.BlockSpec((1, 8, S), lambda b, n: (b, 0, 0)),
        ],
        out_specs=[pl.BlockSpec((1, blk1, 1), lambda b, n: (b, n, 0))
                   for _ in range(3)],
        out_shape=[jax.ShapeDtypeStruct((B, N, 1), jnp.int32)
                   for _ in range(3)],
        compiler_params=pltpu.CompilerParams(
            dimension_semantics=("arbitrary", "arbitrary")),
    )(xyz1p, xyz2t)

    ga, gb, gc, xa, xb, xc = _sc_gather(
        i0.reshape(BN), i1.reshape(BN), i2.reshape(BN), tabf, tabx,
        bn=BN, c2=C2)

    x1f = jnp.pad(xyz1p, ((0, 0), (0, 0), (0, 120))).reshape(BN, 128)
    p1f = points1.reshape(BN, C1)
    nblk2 = BN // blk2
    cspec = pl.BlockSpec((blk2, C2), lambda i: (i, 0))
    xspec = pl.BlockSpec((blk2, 128), lambda i: (i, 0))
    h1, stats1 = pl.pallas_call(
        _stage2_body,
        grid=(nblk2,),
        in_specs=[
            pl.BlockSpec((blk2, 128), lambda i: (i, 0)),
            pl.BlockSpec((blk2, C1), lambda i: (i, 0)),
            cspec, cspec, cspec, xspec, xspec, xspec,
            pl.BlockSpec((C1, C), lambda i: (0, 0)),
            pl.BlockSpec((C2, C), lambda i: (0, 0)),
            pl.BlockSpec((1, C), lambda i: (0, 0)),
        ],
        out_specs=[
            pl.BlockSpec((blk2, C), lambda i: (i, 0)),
            pl.BlockSpec((8, C), lambda i: (0, 0)),
        ],
        out_shape=[
            jax.ShapeDtypeStruct((BN, C), jnp.float32),
            jax.ShapeDtypeStruct((8, C), jnp.float32),
        ],
        compiler_params=pltpu.CompilerParams(
            dimension_semantics=("arbitrary",)),
    )(x1f, p1f, ga, gb, gc, xa, xb, xc, W1a, W1b, b1r)

    cnt = jnp.float32(BN)
    mean1 = stats1[0:1] / cnt
    var1 = stats1[1:2] / cnt - mean1 * mean1
    sc1v = g1.reshape(1, C) / jnp.sqrt(var1 + 1e-5)
    sh1v = be1.reshape(1, C) - mean1 * sc1v

    h2, stats2 = pl.pallas_call(
        _stage3_body,
        grid=(nblk2,),
        in_specs=[
            pl.BlockSpec((blk2, C), lambda i: (i, 0)),
            pl.BlockSpec((1, C), lambda i: (0, 0)),
            pl.BlockSpec((1, C), lambda i: (0, 0)),
            pl.BlockSpec((C, C), lambda i: (0, 0)),
            pl.BlockSpec((1, C), lambda i: (0, 0)),
        ],
        out_specs=[
            pl.BlockSpec((blk2, C), lambda i: (i, 0)),
            pl.BlockSpec((8, C), lambda i: (0, 0)),
        ],
        out_shape=[
            jax.ShapeDtypeStruct((BN, C), jnp.float32),
            jax.ShapeDtypeStruct((8, C), jnp.float32),
        ],
        compiler_params=pltpu.CompilerParams(
            dimension_semantics=("arbitrary",)),
    )(h1, sc1v, sh1v, W2, b2r)

    mean2 = stats2[0:1] / cnt
    var2 = stats2[1:2] / cnt - mean2 * mean2
    sc2v = g2.reshape(1, C) / jnp.sqrt(var2 + 1e-5)
    sh2v = be2.reshape(1, C) - mean2 * sc2v

    out = pl.pallas_call(
        _stage4_body,
        grid=(nblk2,),
        in_specs=[
            pl.BlockSpec((blk2, C), lambda i: (i, 0)),
            pl.BlockSpec((1, C), lambda i: (0, 0)),
            pl.BlockSpec((1, C), lambda i: (0, 0)),
        ],
        out_specs=pl.BlockSpec((blk2, C), lambda i: (i, 0)),
        out_shape=jax.ShapeDtypeStruct((BN, C), jnp.float32),
    )(h2, sc2v, sh2v)

    return out.reshape(B, N, C)
